# baseline (speedup 1.0000x reference)
import jax
import jax.numpy as jnp
from jax import lax
from jax.experimental import pallas as pl
from jax.experimental.pallas import tpu as pltpu
from jax.experimental.pallas import tpu_sc as plsc

D = 128
DC = D // 16
BATCH = 4096
K = 10
NS = 5
NCH = K + NS
NPAIR = NCH - 1
NW = 32
BPW = BATCH // NW
SUB = 8
GIDX = NCH * SUB
NG = BPW // SUB


def _tree_sum(vs):
    vs = list(vs)
    while len(vs) > 1:
        nxt = [vs[i] + vs[i + 1] for i in range(0, len(vs) - 1, 2)]
        if len(vs) % 2:
            nxt.append(vs[-1])
        vs = nxt
    return vs[0]


def _sc_body(walks_hbm, neg_hbm, emb_hbm, out_hbm,
             walks_v, neg_v, idx_v, xbuf, accflat, dots_v, sem0, sem1):
    c = lax.axis_index("c")
    s = lax.axis_index("s")
    wid = s * 2 + c
    base = wid * BPW
    pltpu.sync_copy(walks_hbm.at[pl.ds(base, BPW), :], walks_v)
    pltpu.sync_copy(neg_hbm.at[pl.ds(base, BPW), :], neg_v)

    lane = lax.iota(jnp.int32, 16)
    straddle = jnp.where(lane >= SUB, GIDX - SUB, 0) + lane

    for gp in range(NG // 2):
        rows = lane + gp * 16
        for k in range(NCH):
            if k < K:
                vec = plsc.load_gather(walks_v, [rows, jnp.full((16,), k, jnp.int32)])
            else:
                vec = plsc.load_gather(neg_v, [rows, jnp.full((16,), k - K, jnp.int32)])
            plsc.store_scatter(
                idx_v, [(gp * 2) * GIDX + k * SUB + straddle], vec)

    sems = (sem0, sem1)

    def fire(g, buf):
        return pltpu.async_copy(
            emb_hbm.at[idx_v.at[pl.ds(g * GIDX, GIDX)]],
            xbuf.at[buf], sems[buf])

    def compute(g, buf, half):
        def row_body(r, carry):
            x0c = [xbuf[buf, r, pl.ds(cc * 16, 16)] for cc in range(DC)]
            for k in range(1, NCH):
                acc = _tree_sum(
                    [x0c[cc] * xbuf[buf, k * SUB + r, pl.ds(cc * 16, 16)]
                     for cc in range(DC)])
                plsc.store_scatter(
                    accflat,
                    [lane * 16 + ((k - 1) * 256 + half * SUB + r)], acc)
            return carry

        lax.fori_loop(0, SUB, row_body, 0, unroll=False)

    h = fire(0, 0)
    for g in range(NG):
        nxt = None
        if g + 1 < NG:
            nxt = fire(g + 1, (g + 1) % 2)
        h.wait()
        compute(g, g % 2, g % 2)
        if g % 2 == 1:
            gp = g // 2

            def red_body(k, carry):
                dot = _tree_sum(
                    [accflat[pl.ds(k * 256 + l * 16, 16)] for l in range(16)])
                dots_v[k, pl.ds(gp * 16, 16)] = dot
                return carry

            lax.fori_loop(0, NPAIR, red_body, 0, unroll=False)
        h = nxt

    pltpu.sync_copy(dots_v, out_hbm.at[:, pl.ds(base, BPW)])


@jax.jit
def _sc_dots(walks, neg_samples, embedding):
    mesh = plsc.VectorSubcoreMesh(core_axis_name="c", subcore_axis_name="s")
    return pl.kernel(
        _sc_body,
        out_type=jax.ShapeDtypeStruct((NPAIR, BATCH), jnp.float32),
        mesh=mesh,
        compiler_params=pltpu.CompilerParams(needs_layout_passes=False),
        scratch_types=[
            pltpu.VMEM((BPW, K), jnp.int32),
            pltpu.VMEM((BPW, NS), jnp.int32),
            pltpu.VMEM((NG * GIDX,), jnp.int32),
            pltpu.VMEM((2, GIDX, D), jnp.float32),
            pltpu.VMEM((NPAIR * 256,), jnp.float32),
            pltpu.VMEM((NPAIR, BPW), jnp.float32),
            pltpu.SemaphoreType.DMA,
            pltpu.SemaphoreType.DMA,
        ],
    )(walks, neg_samples, embedding)


def _tc_loss_body(d_ref, out_ref):
    dots = d_ref[...]
    pos = dots[: K - 1]
    neg = dots[K - 1:]
    pos_loss = -jnp.log(jax.nn.sigmoid(pos) + 1e-08)
    neg_loss = -jnp.log(1.0 - jax.nn.sigmoid(neg) + 1e-08)
    out_ref[0, 0] = (jnp.sum(pos_loss) / (BATCH * (K - 1))
                     + jnp.sum(neg_loss) / (BATCH * NS))


@jax.jit
def _tc_loss(dots):
    out = pl.pallas_call(
        _tc_loss_body,
        out_specs=pl.BlockSpec(memory_space=pltpu.SMEM),
        out_shape=jax.ShapeDtypeStruct((1, 1), jnp.float32),
    )(dots)
    return out[0, 0]


def kernel(walks, neg_samples, embedding):
    dots = _sc_dots(walks.astype(jnp.int32), neg_samples.astype(jnp.int32),
                    embedding)
    return _tc_loss(dots)


# speedup vs baseline: 1.3001x; 1.3001x over previous
import jax
import jax.numpy as jnp
from jax import lax
from jax.experimental import pallas as pl
from jax.experimental.pallas import tpu as pltpu
from jax.experimental.pallas import tpu_sc as plsc

D = 128
DC = D // 16
BATCH = 4096
K = 10
NS = 5
NCH = K + NS
NPAIR = NCH - 1
NW = 32
BPW = BATCH // NW
SUB = 8
GIDX = NCH * SUB
NG = BPW // SUB
NROW = BATCH * NCH // 128


def _tree_sum(vs):
    vs = list(vs)
    while len(vs) > 1:
        nxt = [vs[i] + vs[i + 1] for i in range(0, len(vs) - 1, 2)]
        if len(vs) % 2:
            nxt.append(vs[-1])
        vs = nxt
    return vs[0]


def _sc_body(idx_hbm, emb_hbm, out_hbm,
             raw_v, idx_v, xbuf, accflat, dots_v, sem0, sem1):
    c = lax.axis_index("c")
    s = lax.axis_index("s")
    wid = s * 2 + c
    base = wid * BPW
    pltpu.sync_copy(idx_hbm.at[pl.ds(base * NCH, BPW * NCH)], raw_v)

    lane = lax.iota(jnp.int32, 16)
    straddle = jnp.where(lane >= SUB, GIDX - SUB, 0) + lane

    def build_body(gp, carry):
        rows16 = lane + gp * 16

        def k_body(k, carry2):
            p = rows16 * NCH + k
            vec = plsc.load_gather(raw_v, [p])
            plsc.store_scatter(
                idx_v, [(gp * 2) * GIDX + k * SUB + straddle], vec)
            return carry2

        return lax.fori_loop(0, NCH, k_body, carry, unroll=False)

    lax.fori_loop(0, NG // 2, build_body, 0, unroll=False)

    def fire(g, buf):
        return pltpu.async_copy(
            emb_hbm.at[idx_v.at[pl.ds(g * GIDX, GIDX)]],
            xbuf.at[buf], sems[buf])

    sems = (sem0, sem1)

    def wait(buf):
        pltpu.make_async_copy(
            emb_hbm.at[idx_v.at[pl.ds(0, GIDX)]], xbuf.at[buf],
            sems[buf]).wait()

    def compute(buf, half):
        def row_body(r, carry):
            x0c = [xbuf[buf, r, pl.ds(cc * 16, 16)] for cc in range(DC)]
            for k in range(1, NCH):
                acc = _tree_sum(
                    [x0c[cc] * xbuf[buf, k * SUB + r, pl.ds(cc * 16, 16)]
                     for cc in range(DC)])
                plsc.store_scatter(
                    accflat,
                    [lane * 16 + ((k - 1) * 256 + half * SUB + r)], acc)
            return carry

        lax.fori_loop(0, SUB, row_body, 0, unroll=False)

    fire(0, 0)
    fire(1, 1)

    def pair_body(gp, carry):
        wait(0)
        compute(0, 0)

        @pl.when(gp < NG // 2 - 1)
        def _():
            fire_dyn(2 * gp + 2, 0)

        wait(1)
        compute(1, 1)

        @pl.when(gp < NG // 2 - 1)
        def _():
            fire_dyn(2 * gp + 3, 1)

        def red_body(k, carry2):
            dot = _tree_sum(
                [accflat[pl.ds(k * 256 + l * 16, 16)] for l in range(16)])
            dots_v[k, pl.ds(gp * 16, 16)] = dot
            return carry2

        lax.fori_loop(0, NPAIR, red_body, 0, unroll=False)
        return carry

    def fire_dyn(g, buf):
        pltpu.async_copy(
            emb_hbm.at[idx_v.at[pl.ds(g * GIDX, GIDX)]],
            xbuf.at[buf], sems[buf])

    lax.fori_loop(0, NG // 2, pair_body, 0, unroll=False)

    pltpu.sync_copy(dots_v, out_hbm.at[:, pl.ds(base, BPW)])


@jax.jit
def _sc_dots(idx_cat, embedding):
    mesh = plsc.VectorSubcoreMesh(core_axis_name="c", subcore_axis_name="s")
    return pl.kernel(
        _sc_body,
        out_type=jax.ShapeDtypeStruct((NPAIR, BATCH), jnp.float32),
        mesh=mesh,
        compiler_params=pltpu.CompilerParams(needs_layout_passes=False),
        scratch_types=[
            pltpu.VMEM((BPW * NCH,), jnp.int32),
            pltpu.VMEM((NG * GIDX,), jnp.int32),
            pltpu.VMEM((2, GIDX, D), jnp.float32),
            pltpu.VMEM((NPAIR * 256,), jnp.float32),
            pltpu.VMEM((NPAIR, BPW), jnp.float32),
            pltpu.SemaphoreType.DMA,
            pltpu.SemaphoreType.DMA,
        ],
    )(idx_cat, embedding)


def _tc_loss_body(d_ref, out_ref):
    dots = d_ref[...]
    pos = dots[: K - 1]
    neg = dots[K - 1:]
    pos_loss = -jnp.log(jax.nn.sigmoid(pos) + 1e-08)
    neg_loss = -jnp.log(1.0 - jax.nn.sigmoid(neg) + 1e-08)
    out_ref[0, 0] = (jnp.sum(pos_loss) / (BATCH * (K - 1))
                     + jnp.sum(neg_loss) / (BATCH * NS))


@jax.jit
def _tc_loss(dots):
    out = pl.pallas_call(
        _tc_loss_body,
        out_specs=pl.BlockSpec(memory_space=pltpu.SMEM),
        out_shape=jax.ShapeDtypeStruct((1, 1), jnp.float32),
    )(dots)
    return out[0, 0]


def kernel(walks, neg_samples, embedding):
    idx_cat = jnp.concatenate(
        [walks.astype(jnp.int32), neg_samples.astype(jnp.int32)],
        axis=1).reshape(-1)
    dots = _sc_dots(idx_cat, embedding)
    return _tc_loss(dots)
